# baseline (speedup 1.0000x reference)
import jax
import jax.numpy as jnp
from jax import lax
from jax.experimental import pallas as pl
from jax.experimental.pallas import tpu as pltpu
from jax.experimental.pallas import tpu_sc as plsc

_B, _N = 1024, 65536
_K = 32
_LANES = 16


def _sc_body(x_hbm, out_hbm, buf, out_v, sem):
    nc = plsc.get_sparse_core_info().num_cores
    wid = lax.axis_index("s") * nc + lax.axis_index("c")
    rows = _B // (nc * 16)
    r0 = wid * rows

    copies = [
        pltpu.make_async_copy(
            x_hbm.at[pl.ds(r0, rows), pl.ds(2048 * i, _LANES)],
            buf.at[i],
            sem,
        )
        for i in range(_K)
    ]
    for cp in copies:
        cp.start()
    for cp in copies:
        cp.wait()

    lane = lax.iota(jnp.int32, _LANES)
    zeros = jnp.zeros((_LANES,), jnp.int32)

    def extract(t, carry):
        b_loc = t >> 1
        i = (t & 1) * _LANES + lane
        vals = plsc.load_gather(
            buf, [i, jnp.full((_LANES,), b_loc, jnp.int32), zeros]
        )
        out_v[b_loc, pl.ds((t & 1) * _LANES, _LANES)] = vals
        return carry

    lax.fori_loop(0, 2 * rows, extract, 0)

    pltpu.sync_copy(out_v, out_hbm.at[pl.ds(r0, rows), :])


def kernel(x):
    mesh = plsc.VectorSubcoreMesh(core_axis_name="c", subcore_axis_name="s")
    return pl.kernel(
        _sc_body,
        mesh=mesh,
        out_type=jax.ShapeDtypeStruct((_B, _K), jnp.float32),
        scratch_types=[
            pltpu.VMEM((_K, _B // 32, _LANES), jnp.float32),
            pltpu.VMEM((_B // 32, _K), jnp.float32),
            pltpu.SemaphoreType.DMA,
        ],
        compiler_params=pltpu.CompilerParams(
            needs_layout_passes=False, use_tc_tiling_on_sc=False
        ),
    )(x)


# speedup vs baseline: 8.6964x; 8.6964x over previous
import jax
import jax.numpy as jnp
from jax.experimental import pallas as pl


def _body(x_ref, o_ref):
    c = pl.program_id(0)

    @pl.when(c == 0)
    def _():
        o_ref[...] = jnp.zeros_like(o_ref)

    col = jax.lax.broadcasted_iota(jnp.int32, o_ref.shape, 1)
    o_ref[...] = jnp.where(col == c, x_ref[:, 0:1], o_ref[...])


def kernel(x):
    return pl.pallas_call(
        _body,
        grid=(32,),
        in_specs=[pl.BlockSpec((1024, 128), lambda c: (0, 16 * c))],
        out_specs=pl.BlockSpec((1024, 32), lambda c: (0, 0)),
        out_shape=jax.ShapeDtypeStruct((1024, 32), jnp.float32),
    )(x)


# speedup vs baseline: 15.4758x; 1.7796x over previous
import jax
import jax.numpy as jnp
from jax.experimental import pallas as pl


def _body(*refs):
    o_ref = refs[-1]
    o_ref[...] = jnp.concatenate([r[:, 0:1] for r in refs[:-1]], axis=1)


def _spec(i):
    return pl.BlockSpec((1024, 128), lambda _, i=i: (0, 16 * i))


def kernel(x):
    return pl.pallas_call(
        _body,
        grid=(1,),
        in_specs=[_spec(i) for i in range(32)],
        out_specs=pl.BlockSpec((1024, 32), lambda _: (0, 0)),
        out_shape=jax.ShapeDtypeStruct((1024, 32), jnp.float32),
    )(*([x] * 32))
